# baseline (speedup 1.0000x reference)
import functools

import jax
import jax.numpy as jnp
from jax import lax
from jax.experimental import pallas as pl
from jax.experimental.pallas import tpu as pltpu
from jax.experimental.pallas import tpu_sc as plsc

NUM_EMB = 1000000
D = 64
B = 16384 * 20
NC = 2
NS = 16
NW = NC * NS
B_PER_W = B // NW
CHUNK = 128
CHUNKS = B_PER_W // CHUNK


def _emb_kernel(x_hbm, tab_hbm, out_hbm, idx_v, buf0, buf1, sem0, sem1):
    wid = lax.axis_index("s") * NC + lax.axis_index("c")
    pltpu.sync_copy(x_hbm.at[wid], idx_v)

    def gather(c, buf, sem):
        pltpu.async_copy(tab_hbm.at[idx_v.at[c]], buf, sem)

    def wait(buf, sem):
        pltpu.make_async_copy(tab_hbm.at[idx_v.at[0]], buf, sem).wait()

    gather(0, buf0, sem0)
    gather(1, buf1, sem1)

    @pl.loop(0, CHUNKS, step=2)
    def _(c):
        wait(buf0, sem0)

        @pl.when(c + 2 < CHUNKS)
        def _():
            gather(c + 2, buf0, sem0)

        pltpu.sync_copy(buf0, out_hbm.at[wid, c])
        wait(buf1, sem1)

        @pl.when(c + 3 < CHUNKS)
        def _():
            gather(c + 3, buf1, sem1)

        pltpu.sync_copy(buf1, out_hbm.at[wid, c + 1])


@jax.jit
def _emb(x2d, weight):
    mesh = plsc.VectorSubcoreMesh(core_axis_name="c", subcore_axis_name="s")
    f = functools.partial(
        pl.kernel,
        mesh=mesh,
        out_type=jax.ShapeDtypeStruct((NW, CHUNKS, CHUNK, D), jnp.float32),
        scratch_types=[
            pltpu.VMEM((CHUNKS, CHUNK), jnp.int32),
            pltpu.VMEM((CHUNK, D), jnp.float32),
            pltpu.VMEM((CHUNK, D), jnp.float32),
            pltpu.SemaphoreType.DMA,
            pltpu.SemaphoreType.DMA,
        ],
        compiler_params=pltpu.CompilerParams(use_tc_tiling_on_sc=False),
    )(_emb_kernel)
    return f(x2d, weight)


def kernel(x, weight):
    x2d = x.astype(jnp.int32).reshape(NW, CHUNKS, CHUNK)
    out = _emb(x2d, weight)
    return out.reshape(x.shape[0], x.shape[1], D)


# speedup vs baseline: 1.0005x; 1.0005x over previous
import functools

import jax
import jax.numpy as jnp
from jax import lax
from jax.experimental import pallas as pl
from jax.experimental.pallas import tpu as pltpu
from jax.experimental.pallas import tpu_sc as plsc

NUM_EMB = 1000000
D = 64
B = 16384 * 20
NC = 2
NS = 16
NW = NC * NS
B_PER_W = B // NW
CHUNK = 128
CHUNKS = B_PER_W // CHUNK
K = 8
GROUPS = CHUNKS // K


def _emb_kernel(x_hbm, tab_hbm, out_hbm, idx_v, bufs, gsems, wsems):
    wid = lax.axis_index("s") * NC + lax.axis_index("c")
    pltpu.sync_copy(x_hbm.at[wid], idx_v)

    def gather(c, b):
        pltpu.async_copy(tab_hbm.at[idx_v.at[c]], bufs.at[b], gsems.at[b])

    for b in range(K):
        gather(b, b)

    @pl.loop(0, GROUPS)
    def _(g):
        c0 = g * K
        for b in range(K):
            pltpu.make_async_copy(
                tab_hbm.at[idx_v.at[0]], bufs.at[b], gsems.at[b]
            ).wait()
            pltpu.async_copy(bufs.at[b], out_hbm.at[wid, c0 + b], wsems.at[b])
        @pl.when(g + 1 < GROUPS)
        def _():
            for b in range(K):
                pltpu.make_async_copy(
                    bufs.at[b], out_hbm.at[wid, 0], wsems.at[b]
                ).wait()
                gather(c0 + K + b, b)

    for b in range(K):
        pltpu.make_async_copy(bufs.at[b], out_hbm.at[wid, 0], wsems.at[b]).wait()


@jax.jit
def _emb(x2d, weight):
    mesh = plsc.VectorSubcoreMesh(core_axis_name="c", subcore_axis_name="s")
    f = functools.partial(
        pl.kernel,
        mesh=mesh,
        out_type=jax.ShapeDtypeStruct((NW, CHUNKS, CHUNK, D), jnp.float32),
        scratch_types=[
            pltpu.VMEM((CHUNKS, CHUNK), jnp.int32),
            pltpu.VMEM((K, CHUNK, D), jnp.float32),
            pltpu.SemaphoreType.DMA((K,)),
            pltpu.SemaphoreType.DMA((K,)),
        ],
        compiler_params=pltpu.CompilerParams(use_tc_tiling_on_sc=False),
    )(_emb_kernel)
    return f(x2d, weight)


def kernel(x, weight):
    x2d = x.astype(jnp.int32).reshape(NW, CHUNKS, CHUNK)
    out = _emb(x2d, weight)
    return out.reshape(x.shape[0], x.shape[1], D)


# speedup vs baseline: 1.0908x; 1.0903x over previous
import functools

import jax
import jax.numpy as jnp
from jax import lax
from jax.experimental import pallas as pl
from jax.experimental.pallas import tpu as pltpu
from jax.experimental.pallas import tpu_sc as plsc

NUM_EMB = 1000000
D = 64
B = 16384 * 20
NC = 2
NS = 16
NW = NC * NS
B_PER_W = B // NW
CHUNK = 128
CHUNKS = B_PER_W // CHUNK
K = 8
GROUPS = CHUNKS // K

TBLK = 512
TGRID = -(-NUM_EMB // (2 * TBLK))
W2_ROWS = TGRID * TBLK


def _t_body(lo_ref, hi_ref, out_ref):
    out_ref[:, 0:64] = lo_ref[...].T
    out_ref[:, 64:128] = hi_ref[...].T


def _emb_kernel(x_hbm, tab_hbm, out_hbm, idx_v, bufs, gsems, wsems):
    wid = lax.axis_index("s") * NC + lax.axis_index("c")
    pltpu.sync_copy(x_hbm.at[wid], idx_v)

    def gather(c, b):
        pltpu.async_copy(tab_hbm.at[idx_v.at[c]], bufs.at[b], gsems.at[b])

    for b in range(K):
        gather(b, b)

    @pl.loop(0, GROUPS)
    def _(g):
        c0 = g * K
        for b in range(K):
            pltpu.make_async_copy(
                tab_hbm.at[idx_v.at[0]], bufs.at[b], gsems.at[b]
            ).wait()
            pltpu.async_copy(bufs.at[b], out_hbm.at[wid, c0 + b], wsems.at[b])

        @pl.when(g + 1 < GROUPS)
        def _():
            for b in range(K):
                pltpu.make_async_copy(
                    bufs.at[b], out_hbm.at[wid, 0], wsems.at[b]
                ).wait()
                gather(c0 + K + b, b)

    for b in range(K):
        pltpu.make_async_copy(bufs.at[b], out_hbm.at[wid, 0], wsems.at[b]).wait()


@jax.jit
def _emb(x2d, weight):
    mesh = plsc.VectorSubcoreMesh(core_axis_name="c", subcore_axis_name="s")

    w2 = pl.pallas_call(
        _t_body,
        grid=(TGRID,),
        in_specs=[
            pl.BlockSpec((64, TBLK), lambda i: (0, 2 * i)),
            pl.BlockSpec((64, TBLK), lambda i: (0, 2 * i + 1)),
        ],
        out_specs=pl.BlockSpec((TBLK, 128), lambda i: (i, 0)),
        out_shape=jax.ShapeDtypeStruct((W2_ROWS, 128), jnp.float32),
    )(weight.T, weight.T)

    g_fn = functools.partial(
        pl.kernel,
        mesh=mesh,
        out_type=jax.ShapeDtypeStruct((NW, CHUNKS, CHUNK, D), jnp.float32),
        scratch_types=[
            pltpu.VMEM((CHUNKS, CHUNK), jnp.int32),
            pltpu.VMEM((K, CHUNK, D), jnp.float32),
            pltpu.SemaphoreType.DMA((K,)),
            pltpu.SemaphoreType.DMA((K,)),
        ],
        compiler_params=pltpu.CompilerParams(use_tc_tiling_on_sc=False),
    )(_emb_kernel)
    out = g_fn(x2d, w2.reshape(2 * W2_ROWS, D))
    return out


def kernel(x, weight):
    xi = x.astype(jnp.int32)
    blk = xi >> 10
    off = xi & 1023
    xr = (blk << 10) + 2 * (off & 511) + (off >> 9)
    x2d = xr.reshape(NW, CHUNKS, CHUNK)
    out = _emb(x2d, weight)
    return out.reshape(x.shape[0], x.shape[1], D)
